# baseline (speedup 1.0000x reference)
import functools

import jax
import jax.numpy as jnp
from jax import lax
from jax.experimental import pallas as pl
from jax.experimental.pallas import tpu as pltpu
from jax.experimental.pallas import tpu_sc as plsc


def _proj_body(t_ref, w_ref, b_ref, out_ref):
    out_ref[...] = lax.dot_general(
        t_ref[...], w_ref[...], (((1,), (1,)), ((), ())),
        preferred_element_type=jnp.float32,
    ) + b_ref[...]


def _project_table(table, W, b):
    V, D = table.shape
    BLK = 200
    return pl.pallas_call(
        _proj_body,
        grid=(V // BLK,),
        in_specs=[
            pl.BlockSpec((BLK, D), lambda i: (i, 0)),
            pl.BlockSpec((D, D), lambda i: (0, 0)),
            pl.BlockSpec((1, D), lambda i: (0, 0)),
        ],
        out_specs=pl.BlockSpec((BLK, D), lambda i: (i, 0)),
        out_shape=jax.ShapeDtypeStruct((V, D), jnp.float32),
    )(table, W, b.reshape(1, D))


def _relayout_body(Ll, Lp, BB, in_ref, out_ref):
    x = in_ref[...].reshape(BB, Lp, in_ref.shape[-1])
    out_ref[...] = x[:, :Ll, :]


def _relayout(flat, Bb, Ll, Lp, D):
    BB = 8
    return pl.pallas_call(
        functools.partial(_relayout_body, Ll, Lp, BB),
        grid=(Bb // BB,),
        in_specs=[pl.BlockSpec((BB * Lp, D), lambda i: (i, 0))],
        out_specs=pl.BlockSpec((BB, Ll, D), lambda i: (i, 0, 0)),
        out_shape=jax.ShapeDtypeStruct((Bb, Ll, D), jnp.float32),
    )(flat)


def _make_gather(ntok, D, CH):
    info = plsc.get_sparse_core_info()
    NC, NS = info.num_cores, info.num_subcores
    NW = NC * NS
    tpw = ntok // NW
    nch = tpw // CH
    mesh = plsc.VectorSubcoreMesh(core_axis_name="c", subcore_axis_name="s")

    @functools.partial(
        pl.kernel,
        out_type=jax.ShapeDtypeStruct((ntok, D), jnp.float32),
        mesh=mesh,
        scratch_types=[
            pltpu.VMEM((tpw,), jnp.int32),
            pltpu.VMEM((CH, D), jnp.float32),
            pltpu.VMEM((CH, D), jnp.float32),
            pltpu.SemaphoreType.DMA,
            pltpu.SemaphoreType.DMA,
            pltpu.SemaphoreType.DMA,
            pltpu.SemaphoreType.DMA,
        ],
    )
    def gather(p_hbm, idx_hbm, out_hbm, idx_v, buf0, buf1, gs0, gs1, os0, os1):
        wid = lax.axis_index("s") * NC + lax.axis_index("c")
        base = wid * tpw
        pltpu.sync_copy(idx_hbm.at[pl.ds(base, tpw)], idx_v)

        def g_copy(i, buf, sem):
            return pltpu.make_async_copy(
                p_hbm.at[idx_v.at[pl.ds(i * CH, CH)]], buf, sem)

        def o_copy(i, buf, sem):
            return pltpu.make_async_copy(
                buf, out_hbm.at[pl.ds(base + i * CH, CH)], sem)

        g_copy(0, buf0, gs0).start()
        G = nch // 2

        def body(g, carry):
            c0 = 2 * g
            g_copy(c0, buf0, gs0).wait()
            @pl.when(g > 0)
            def _():
                o_copy(c0 - 1, buf1, os1).wait()
            g_copy(c0 + 1, buf1, gs1).start()
            o_copy(c0, buf0, os0).start()
            g_copy(c0 + 1, buf1, gs1).wait()
            o_copy(c0, buf0, os0).wait()
            @pl.when(g < G - 1)
            def _():
                g_copy(c0 + 2, buf0, gs0).start()
            o_copy(c0 + 1, buf1, os1).start()
            return carry

        lax.fori_loop(0, G, body, 0)
        o_copy(nch - 1, buf1, os1).wait()

    return gather


def kernel(token_ids, table, W, b):
    Bb, Ll = token_ids.shape
    V, D = table.shape
    Lp = 56
    P = _project_table(table, W, b)
    ids = jnp.pad(token_ids.astype(jnp.int32), ((0, 0), (0, Lp - Ll)))
    flat = _make_gather(Bb * Lp, D, Lp)(P, ids.reshape(Bb * Lp))
    return _relayout(flat, Bb, Ll, Lp, D)


# speedup vs baseline: 1.5434x; 1.5434x over previous
import functools

import jax
import jax.numpy as jnp
from jax import lax
from jax.experimental import pallas as pl
from jax.experimental.pallas import tpu as pltpu
from jax.experimental.pallas import tpu_sc as plsc


def _proj_body(t_ref, w_ref, b_ref, out_ref):
    out_ref[...] = lax.dot_general(
        t_ref[...], w_ref[...], (((1,), (1,)), ((), ())),
        preferred_element_type=jnp.float32,
    ) + b_ref[...]


def _project_table(table, W, b):
    V, D = table.shape
    BLK = 200
    return pl.pallas_call(
        _proj_body,
        grid=(V // BLK,),
        in_specs=[
            pl.BlockSpec((BLK, D), lambda i: (i, 0)),
            pl.BlockSpec((D, D), lambda i: (0, 0)),
            pl.BlockSpec((1, D), lambda i: (0, 0)),
        ],
        out_specs=pl.BlockSpec((BLK, D), lambda i: (i, 0)),
        out_shape=jax.ShapeDtypeStruct((V, D), jnp.float32),
    )(table, W, b.reshape(1, D))


def _relayout_body(Ll, Lp, BB, in_ref, out_ref):
    x = in_ref[...].reshape(BB, Lp, in_ref.shape[-1])
    out_ref[...] = x[:, :Ll, :]


def _relayout(flat, Bb, Ll, Lp, D):
    BB = 8
    return pl.pallas_call(
        functools.partial(_relayout_body, Ll, Lp, BB),
        grid=(Bb // BB,),
        in_specs=[pl.BlockSpec((BB * Lp, D), lambda i: (i, 0))],
        out_specs=pl.BlockSpec((BB, Ll, D), lambda i: (i, 0, 0)),
        out_shape=jax.ShapeDtypeStruct((Bb, Ll, D), jnp.float32),
    )(flat)


def _make_gather(ntok, D, CH):
    info = plsc.get_sparse_core_info()
    NC, NS = info.num_cores, info.num_subcores
    NW = NC * NS
    tpw = ntok // NW
    nch = tpw // CH
    mesh = plsc.VectorSubcoreMesh(core_axis_name="c", subcore_axis_name="s")

    @functools.partial(
        pl.kernel,
        out_type=jax.ShapeDtypeStruct((ntok, D), jnp.float32),
        mesh=mesh,
        scratch_types=[
            pltpu.VMEM((tpw,), jnp.int32),
            pltpu.VMEM((CH, D), jnp.float32),
            pltpu.VMEM((CH, D), jnp.float32),
            pltpu.SemaphoreType.DMA,
            pltpu.SemaphoreType.DMA,
            pltpu.SemaphoreType.DMA,
            pltpu.SemaphoreType.DMA,
        ],
    )
    def gather(p_hbm, idx_hbm, out_hbm, idx_v, buf0, buf1, gs0, gs1, os0, os1):
        wid = lax.axis_index("s") * NC + lax.axis_index("c")
        base = wid * tpw
        pltpu.sync_copy(idx_hbm.at[pl.ds(base, tpw)], idx_v)

        def g_copy(i, buf, sem):
            return pltpu.make_async_copy(
                p_hbm.at[idx_v.at[pl.ds(i * CH, CH)]], buf, sem)

        def o_copy(i, buf, sem):
            return pltpu.make_async_copy(
                buf, out_hbm.at[pl.ds(base + i * CH, CH)], sem)

        g_copy(0, buf0, gs0).start()
        G = nch // 2

        def body(g, carry):
            c0 = 2 * g
            g_copy(c0, buf0, gs0).wait()
            @pl.when(g > 0)
            def _():
                o_copy(c0 - 1, buf1, os1).wait()
            g_copy(c0 + 1, buf1, gs1).start()
            o_copy(c0, buf0, os0).start()
            g_copy(c0 + 1, buf1, gs1).wait()
            o_copy(c0, buf0, os0).wait()
            @pl.when(g < G - 1)
            def _():
                g_copy(c0 + 2, buf0, gs0).start()
            o_copy(c0 + 1, buf1, os1).start()
            return carry

        lax.fori_loop(0, G, body, 0)
        o_copy(nch - 1, buf1, os1).wait()

    return gather


def kernel(token_ids, table, W, b):
    Bb, Ll = token_ids.shape
    V, D = table.shape
    ntok = Bb * Ll
    P = _project_table(table, W, b)
    ids = token_ids.reshape(ntok).astype(jnp.int32)
    flat = _make_gather(ntok, D, 40)(P, ids)
    return flat.reshape(Bb, Ll, D) * jnp.float32(1.0 + 2.0 ** -23)


# speedup vs baseline: 5.3006x; 3.4343x over previous
import functools

import jax
import jax.numpy as jnp
from jax import lax
from jax.experimental import pallas as pl
from jax.experimental.pallas import tpu as pltpu
from jax.experimental.pallas import tpu_sc as plsc


def _proj_body(t_ref, w_ref, b_ref, out_ref):
    out_ref[...] = lax.dot_general(
        t_ref[...], w_ref[...], (((1,), (1,)), ((), ())),
        preferred_element_type=jnp.float32,
    ) + b_ref[...]


def _project_table(table, W, b):
    V, D = table.shape
    BLK = 200
    return pl.pallas_call(
        _proj_body,
        grid=(V // BLK,),
        in_specs=[
            pl.BlockSpec((BLK, D), lambda i: (i, 0)),
            pl.BlockSpec((D, D), lambda i: (0, 0)),
            pl.BlockSpec((1, D), lambda i: (0, 0)),
        ],
        out_specs=pl.BlockSpec((BLK, D), lambda i: (i, 0)),
        out_shape=jax.ShapeDtypeStruct((V, D), jnp.float32),
    )(table, W, b.reshape(1, D))


def _relayout_body(Ll, Lp, BB, in_ref, out_ref):
    x = in_ref[...].reshape(BB, Lp, in_ref.shape[-1])
    out_ref[...] = x[:, :Ll, :]


def _relayout(flat, Bb, Ll, Lp, D):
    BB = 8
    return pl.pallas_call(
        functools.partial(_relayout_body, Ll, Lp, BB),
        grid=(Bb // BB,),
        in_specs=[pl.BlockSpec((BB * Lp, D), lambda i: (i, 0))],
        out_specs=pl.BlockSpec((BB, Ll, D), lambda i: (i, 0, 0)),
        out_shape=jax.ShapeDtypeStruct((Bb, Ll, D), jnp.float32),
    )(flat)


def _make_gather(ntok, D, CH):
    info = plsc.get_sparse_core_info()
    NC, NS = info.num_cores, info.num_subcores
    NW = NC * NS
    tpw = ntok // NW
    nch = tpw // CH
    mesh = plsc.VectorSubcoreMesh(core_axis_name="c", subcore_axis_name="s")

    @functools.partial(
        pl.kernel,
        out_type=jax.ShapeDtypeStruct((ntok, D), jnp.float32),
        mesh=mesh,
        scratch_types=[
            pltpu.VMEM((tpw,), jnp.int32),
            pltpu.VMEM((CH, D), jnp.float32),
            pltpu.VMEM((CH, D), jnp.float32),
            pltpu.SemaphoreType.DMA,
            pltpu.SemaphoreType.DMA,
            pltpu.SemaphoreType.DMA,
            pltpu.SemaphoreType.DMA,
        ],
    )
    def gather(p_hbm, idx_hbm, out_hbm, idx_v, buf0, buf1, gs0, gs1, os0, os1):
        wid = lax.axis_index("s") * NC + lax.axis_index("c")
        base = wid * tpw
        pltpu.sync_copy(idx_hbm.at[pl.ds(base, tpw)], idx_v)

        def g_copy(i, buf, sem):
            return pltpu.make_async_copy(
                p_hbm.at[idx_v.at[pl.ds(i * CH, CH)]], buf, sem)

        def o_copy(i, buf, sem):
            return pltpu.make_async_copy(
                buf, out_hbm.at[pl.ds(base + i * CH, CH)], sem)

        g_copy(0, buf0, gs0).start()
        G = nch // 2

        def body(g, carry):
            c0 = 2 * g
            g_copy(c0, buf0, gs0).wait()
            @pl.when(g > 0)
            def _():
                o_copy(c0 - 1, buf1, os1).wait()
            g_copy(c0 + 1, buf1, gs1).start()
            o_copy(c0, buf0, os0).start()
            g_copy(c0 + 1, buf1, gs1).wait()
            o_copy(c0, buf0, os0).wait()
            @pl.when(g < G - 1)
            def _():
                g_copy(c0 + 2, buf0, gs0).start()
            o_copy(c0 + 1, buf1, os1).start()
            return carry

        lax.fori_loop(0, G, body, 0)
        o_copy(nch - 1, buf1, os1).wait()

    return gather


def kernel(token_ids, table, W, b):
    Bb, Ll = token_ids.shape
    V, D = table.shape
    ntok = Bb * Ll
    P = _project_table(table, W, b)
    ids = token_ids.astype(jnp.int32).T.reshape(ntok)
    flat = _make_gather(ntok, D, 40)(P, ids)
    return flat.reshape(Ll, Bb, D).transpose(1, 0, 2)


# speedup vs baseline: 5.3949x; 1.0178x over previous
import functools

import jax
import jax.numpy as jnp
from jax import lax
from jax.experimental import pallas as pl
from jax.experimental.pallas import tpu as pltpu
from jax.experimental.pallas import tpu_sc as plsc


def _proj_body(t_ref, w_ref, b_ref, out_ref):
    out_ref[...] = lax.dot_general(
        t_ref[...], w_ref[...], (((1,), (1,)), ((), ())),
        preferred_element_type=jnp.float32,
    ) + b_ref[...]


def _project_table(table, W, b):
    V, D = table.shape
    BLK = 200
    return pl.pallas_call(
        _proj_body,
        grid=(V // BLK,),
        in_specs=[
            pl.BlockSpec((BLK, D), lambda i: (i, 0)),
            pl.BlockSpec((D, D), lambda i: (0, 0)),
            pl.BlockSpec((1, D), lambda i: (0, 0)),
        ],
        out_specs=pl.BlockSpec((BLK, D), lambda i: (i, 0)),
        out_shape=jax.ShapeDtypeStruct((V, D), jnp.float32),
    )(table, W, b.reshape(1, D))


def _relayout_body(Ll, Lp, BB, in_ref, out_ref):
    x = in_ref[...].reshape(BB, Lp, in_ref.shape[-1])
    out_ref[...] = x[:, :Ll, :]


def _relayout(flat, Bb, Ll, Lp, D):
    BB = 8
    return pl.pallas_call(
        functools.partial(_relayout_body, Ll, Lp, BB),
        grid=(Bb // BB,),
        in_specs=[pl.BlockSpec((BB * Lp, D), lambda i: (i, 0))],
        out_specs=pl.BlockSpec((BB, Ll, D), lambda i: (i, 0, 0)),
        out_shape=jax.ShapeDtypeStruct((Bb, Ll, D), jnp.float32),
    )(flat)


def _make_gather(ntok, D, CH):
    info = plsc.get_sparse_core_info()
    NC, NS = info.num_cores, info.num_subcores
    NW = NC * NS
    tpw = ntok // NW
    nch = tpw // CH
    mesh = plsc.VectorSubcoreMesh(core_axis_name="c", subcore_axis_name="s")

    @functools.partial(
        pl.kernel,
        out_type=jax.ShapeDtypeStruct((ntok, D), jnp.float32),
        mesh=mesh,
        scratch_types=[
            pltpu.VMEM((tpw,), jnp.int32),
            pltpu.VMEM((CH, D), jnp.float32),
            pltpu.VMEM((CH, D), jnp.float32),
            pltpu.VMEM((CH, D), jnp.float32),
            pltpu.SemaphoreType.DMA,
            pltpu.SemaphoreType.DMA,
            pltpu.SemaphoreType.DMA,
            pltpu.SemaphoreType.DMA,
            pltpu.SemaphoreType.DMA,
            pltpu.SemaphoreType.DMA,
        ],
    )
    def gather(p_hbm, idx_hbm, out_hbm, idx_v,
               buf0, buf1, buf2, gs0, gs1, gs2, os0, os1, os2):
        wid = lax.axis_index("s") * NC + lax.axis_index("c")
        base = wid * tpw
        pltpu.sync_copy(idx_hbm.at[pl.ds(base, tpw)], idx_v)
        bufs = (buf0, buf1, buf2)
        gss = (gs0, gs1, gs2)
        oss = (os0, os1, os2)

        def g_copy(i, s):
            return pltpu.make_async_copy(
                p_hbm.at[idx_v.at[pl.ds(i * CH, CH)]], bufs[s], gss[s])

        def o_copy(i, s):
            return pltpu.make_async_copy(
                bufs[s], out_hbm.at[pl.ds(base + i * CH, CH)], oss[s])

        g_copy(0, 0).start()
        g_copy(1, 1).start()
        ntrip = (nch + 2 + 2) // 3

        def body(k, carry):
            for j in range(3):
                i = 3 * k + j
                sj = j
                sp = (j + 2) % 3

                @pl.when(i < nch)
                def _():
                    g_copy(i, sj).wait()
                @pl.when(jnp.logical_and(i >= 1, i <= nch))
                def _():
                    o_copy(i - 1, sp).wait()
                @pl.when(i + 2 < nch)
                def _():
                    g_copy(i + 2, sp).start()
                @pl.when(i < nch)
                def _():
                    o_copy(i, sj).start()
            return carry

        lax.fori_loop(0, ntrip, body, 0)

    return gather


def kernel(token_ids, table, W, b):
    Bb, Ll = token_ids.shape
    V, D = table.shape
    ntok = Bb * Ll
    P = _project_table(table, W, b)
    ids = token_ids.astype(jnp.int32).T.reshape(ntok)
    flat = _make_gather(ntok, D, 40)(P, ids)
    return flat.reshape(Ll, Bb, D).transpose(1, 0, 2)


# speedup vs baseline: 5.4039x; 1.0017x over previous
import functools

import jax
import jax.numpy as jnp
from jax import lax
from jax.experimental import pallas as pl
from jax.experimental.pallas import tpu as pltpu
from jax.experimental.pallas import tpu_sc as plsc


def _proj_body(t_ref, w_ref, b_ref, out_ref):
    out_ref[...] = lax.dot_general(
        t_ref[...], w_ref[...], (((1,), (1,)), ((), ())),
        preferred_element_type=jnp.float32,
    ) + b_ref[...]


def _project_table(table, W, b):
    V, D = table.shape
    BLK = 200
    return pl.pallas_call(
        _proj_body,
        grid=(V // BLK,),
        in_specs=[
            pl.BlockSpec((BLK, D), lambda i: (i, 0)),
            pl.BlockSpec((D, D), lambda i: (0, 0)),
            pl.BlockSpec((1, D), lambda i: (0, 0)),
        ],
        out_specs=pl.BlockSpec((BLK, D), lambda i: (i, 0)),
        out_shape=jax.ShapeDtypeStruct((V, D), jnp.float32),
    )(table, W, b.reshape(1, D))


def _relayout_body(Ll, Lp, BB, in_ref, out_ref):
    x = in_ref[...].reshape(BB, Lp, in_ref.shape[-1])
    out_ref[...] = x[:, :Ll, :]


def _relayout(flat, Bb, Ll, Lp, D):
    BB = 8
    return pl.pallas_call(
        functools.partial(_relayout_body, Ll, Lp, BB),
        grid=(Bb // BB,),
        in_specs=[pl.BlockSpec((BB * Lp, D), lambda i: (i, 0))],
        out_specs=pl.BlockSpec((BB, Ll, D), lambda i: (i, 0, 0)),
        out_shape=jax.ShapeDtypeStruct((Bb, Ll, D), jnp.float32),
    )(flat)


def _make_gather(ntok, D, CH):
    info = plsc.get_sparse_core_info()
    NC, NS = info.num_cores, info.num_subcores
    NW = NC * NS
    tpw = ntok // NW
    nch = tpw // CH
    mesh = plsc.VectorSubcoreMesh(core_axis_name="c", subcore_axis_name="s")

    @functools.partial(
        pl.kernel,
        out_type=jax.ShapeDtypeStruct((ntok, D), jnp.float32),
        mesh=mesh,
        scratch_types=[
            pltpu.VMEM((tpw,), jnp.int32),
            pltpu.VMEM((CH, D), jnp.float32),
            pltpu.VMEM((CH, D), jnp.float32),
            pltpu.VMEM((CH, D), jnp.float32),
            pltpu.SemaphoreType.DMA,
            pltpu.SemaphoreType.DMA,
            pltpu.SemaphoreType.DMA,
            pltpu.SemaphoreType.DMA,
            pltpu.SemaphoreType.DMA,
            pltpu.SemaphoreType.DMA,
        ],
    )
    def gather(p_hbm, idx_hbm, out_hbm, idx_v,
               buf0, buf1, buf2, gs0, gs1, gs2, os0, os1, os2):
        wid = lax.axis_index("s") * NC + lax.axis_index("c")
        base = wid * tpw
        pltpu.sync_copy(idx_hbm.at[pl.ds(base, tpw)], idx_v)
        bufs = (buf0, buf1, buf2)
        gss = (gs0, gs1, gs2)
        oss = (os0, os1, os2)

        def g_copy(i, s):
            return pltpu.make_async_copy(
                p_hbm.at[idx_v.at[pl.ds(i * CH, CH)]], bufs[s], gss[s])

        def o_copy(i, s):
            return pltpu.make_async_copy(
                bufs[s], out_hbm.at[pl.ds(base + i * CH, CH)], oss[s])

        g_copy(0, 0).start()
        g_copy(1, 1).start()
        ntrip = (nch + 2 + 2) // 3

        def body(k, carry):
            for j in range(3):
                i = 3 * k + j
                sj = j
                sp = (j + 2) % 3

                @pl.when(i < nch)
                def _():
                    g_copy(i, sj).wait()
                @pl.when(jnp.logical_and(i >= 1, i <= nch))
                def _():
                    o_copy(i - 1, sp).wait()
                @pl.when(i + 2 < nch)
                def _():
                    g_copy(i + 2, sp).start()
                @pl.when(i < nch)
                def _():
                    o_copy(i, sj).start()
            return carry

        lax.fori_loop(0, ntrip, body, 0)

    return gather


def kernel(token_ids, table, W, b):
    Bb, Ll = token_ids.shape
    V, D = table.shape
    ntok = Bb * Ll
    P = _project_table(table, W, b)
    ids = token_ids.astype(jnp.int32).T.reshape(ntok)
    flat = _make_gather(ntok, D, 32)(P, ids)
    return flat.reshape(Ll, Bb, D).transpose(1, 0, 2)
